# Initial kernel scaffold; baseline (speedup 1.0000x reference)
import jax
import jax.numpy as jnp
from jax.experimental import pallas as pl


def kernel(x, mgroupdef, e2h_edge_index, h2h_edge_index, h2e_edge_index, e2h_edge_attr, h2h_edge_attr, h2e_edge_attr, era_latlons, h_latlons, params):
    raise NotImplementedError("write your pallas kernel here")


# speedup vs baseline: 1.5325x; 1.5325x over previous
import functools

import jax
import jax.numpy as jnp
from jax import lax
from jax.experimental import pallas as pl
from jax.experimental.pallas import tpu as pltpu
from jax.experimental.pallas import tpu_sc as plsc

NC, NS = 2, 16
NW = NC * NS
DM = 128


def _rup(n, m):
    return (n + m - 1) // m * m


def _mlp(p, xs, *, residual=False, e_new=False, proj=None, br=1024):
    n = xs[0].shape[0]
    dins = [x.shape[1] for x in xs]
    k = len(xs)
    dout = proj[0].shape[1] if proj is not None else DM

    def body(*refs):
        xrefs = refs[:k]
        w1, b1, w2, b2, g, bn = refs[k:k + 6]
        pos = k + 6
        if proj is not None:
            wo, bo = refs[pos:pos + 2]
            pos += 2
        outs = refs[pos:]
        acc = None
        off = 0
        for i in range(k):
            part = jnp.dot(xrefs[i][...], w1[off:off + dins[i], :],
                           preferred_element_type=jnp.float32)
            acc = part if acc is None else acc + part
            off += dins[i]
        h = acc + b1[...]
        h = h * jax.nn.sigmoid(h)
        y = jnp.dot(h, w2[...], preferred_element_type=jnp.float32) + b2[...]
        mu = jnp.mean(y, -1, keepdims=True)
        yc = y - mu
        var = jnp.mean(yc * yc, -1, keepdims=True)
        m = yc * lax.rsqrt(var + 1e-5) * g[...] + bn[...]
        r = xrefs[0][...] + m if residual else m
        if proj is not None:
            r = jnp.dot(r, wo[...], preferred_element_type=jnp.float32) + bo[...]
        outs[0][...] = r
        if e_new:
            outs[1][...] = xrefs[-1][...] + m

    in_specs = [pl.BlockSpec((br, d), lambda i: (i, 0)) for d in dins]
    w_args = [p["W1"], p["b1"].reshape(1, DM), p["W2"], p["b2"].reshape(1, DM),
              p["g"].reshape(1, DM), p["bn"].reshape(1, DM)]
    for w in w_args:
        in_specs.append(pl.BlockSpec(w.shape, lambda i: (0, 0)))
    args = list(xs) + w_args
    if proj is not None:
        wo, bo = proj
        args += [wo, bo.reshape(1, -1)]
        in_specs.append(pl.BlockSpec(wo.shape, lambda i: (0, 0)))
        in_specs.append(pl.BlockSpec((1, dout), lambda i: (0, 0)))
    out_shape = [jax.ShapeDtypeStruct((n, dout), jnp.float32)]
    out_specs = [pl.BlockSpec((br, dout), lambda i: (i, 0))]
    if e_new:
        out_shape.append(jax.ShapeDtypeStruct((n, DM), jnp.float32))
        out_specs.append(pl.BlockSpec((br, DM), lambda i: (i, 0)))
    res = pl.pallas_call(
        body,
        grid=(pl.cdiv(n, br),),
        in_specs=in_specs,
        out_specs=out_specs,
        out_shape=out_shape,
    )(*args)
    return res if e_new else res[0]


def _sc_gather2(ta, tb, ia2, ib2):
    nchunks = ia2.shape[0]
    cpw = nchunks // NW
    e_pad = nchunks * 128
    mesh = plsc.VectorSubcoreMesh(core_axis_name="c", subcore_axis_name="s",
                                  num_cores=NC, num_subcores=NS)

    @functools.partial(
        pl.kernel,
        out_type=(jax.ShapeDtypeStruct((e_pad, DM), jnp.float32),
                  jax.ShapeDtypeStruct((e_pad, DM), jnp.float32)),
        mesh=mesh,
        scratch_types=[pltpu.VMEM((1, 128), jnp.int32),
                       pltpu.VMEM((128, DM), jnp.float32),
                       pltpu.VMEM((1, 128), jnp.int32),
                       pltpu.VMEM((128, DM), jnp.float32),
                       pltpu.SemaphoreType.DMA,
                       pltpu.SemaphoreType.DMA],
    )
    def k(ta_h, tb_h, ia_h, ib_h, oa_h, ob_h, idxa, rowsa, idxb, rowsb, sema, semb):
        wid = lax.axis_index("s") * NC + lax.axis_index("c")
        c0 = wid * cpw

        def chunk(i, carry):
            gi = c0 + i
            pltpu.sync_copy(ia_h.at[gi], idxa)
            pltpu.sync_copy(ib_h.at[gi], idxb)
            cpa = pltpu.async_copy(ta_h.at[idxa.at[0]], rowsa, sema)
            cpb = pltpu.async_copy(tb_h.at[idxb.at[0]], rowsb, semb)
            cpa.wait()
            pltpu.sync_copy(rowsa, oa_h.at[pl.ds(gi * 128, 128)])
            cpb.wait()
            pltpu.sync_copy(rowsb, ob_h.at[pl.ds(gi * 128, 128)])
            return carry

        lax.fori_loop(0, cpw, chunk, 0)

    return k(ta, tb, ia2, ib2)


def _sc_scatter(m2, idx2, n_nodes, n_passes):
    nchunks = idx2.shape[0]
    cpt = nchunks // NS
    r_al = _rup(-(-n_nodes // (NC * n_passes)), 128)
    r_pad = _rup(r_al + 1, 256)
    wb = r_al // NS
    mesh = plsc.VectorSubcoreMesh(core_axis_name="c", subcore_axis_name="s",
                                  num_cores=NC, num_subcores=NS)

    @functools.partial(
        pl.kernel,
        out_type=jax.ShapeDtypeStruct((n_nodes, DM), jnp.float32),
        mesh=mesh,
        scratch_types=[pltpu.VMEM((16, DM), jnp.float32),
                       pltpu.VMEM((1, 128), jnp.int32),
                       pltpu.VMEM((1, 128), jnp.int32),
                       pltpu.VMEM((128, DM), jnp.float32),
                       pltpu.VMEM_SHARED((r_pad, DM), jnp.float32),
                       pltpu.SemaphoreType.DMA],
    )
    def k(m_h, i_h, out_h, zbuf, idxraw, lidx, mrows, shared, sem):
        cid = lax.axis_index("c")
        sid = lax.axis_index("s")
        for rr in range(16):
            for j in range(8):
                zbuf[rr, pl.ds(j * 16, 16)] = jnp.zeros((16,), jnp.float32)
        zb = sid * (r_pad // NS)
        for pss in range(n_passes):
            rs = jnp.minimum((cid * n_passes + pss) * r_al, n_nodes - r_al)
            for z in range(r_pad // 256):
                pltpu.sync_copy(zbuf, shared.at[pl.ds(zb + z * 16, 16)])
            plsc.subcore_barrier()

            def chunk(i, carry):
                gi = sid * cpt + i
                pltpu.sync_copy(i_h.at[gi], idxraw)
                pltpu.sync_copy(m_h.at[pl.ds(gi * 128, 128)], mrows)
                for j in range(8):
                    v = idxraw[0, pl.ds(j * 16, 16)]
                    li = v - rs
                    okm = (li >= 0) & (li < r_al)
                    lidx[0, pl.ds(j * 16, 16)] = jnp.where(okm, li, r_al)
                pltpu.async_copy(mrows, shared.at[lidx.at[0]], sem, add=True).wait()
                return carry

            lax.fori_loop(0, cpt, chunk, 0)
            plsc.subcore_barrier()
            pltpu.sync_copy(shared.at[pl.ds(sid * wb, wb)],
                            out_h.at[pl.ds(rs + sid * wb, wb)])
            plsc.subcore_barrier()

    return k(m2, idx2)


def _pad_rows(a, n_pad):
    e = a.shape[0]
    if e == n_pad:
        return a
    return jnp.concatenate(
        [a, jnp.zeros((n_pad - e,) + a.shape[1:], a.dtype)], axis=0)


def _pad_idx(idx, n_pad, fill):
    e = idx.shape[0]
    if e != n_pad:
        idx = jnp.concatenate(
            [idx, jnp.full((n_pad - e,), fill, jnp.int32)], axis=0)
    return idx.reshape(-1, 1, 128)


def kernel(x, mgroupdef, e2h_edge_index, h2h_edge_index, h2e_edge_index,
           e2h_edge_attr, h2h_edge_attr, h2e_edge_attr,
           era_latlons, h_latlons, params):
    p = params
    bs = x.shape[0]
    n_era = x.shape[2]
    n_h = h_latlons.shape[0]
    e_e2h = e2h_edge_index.shape[1]
    e_h2h = h2h_edge_index.shape[1]
    e_h2e = h2e_edge_index.shape[1]
    e2h_pad = _rup(e_e2h, NW * 128)
    h2h_pad = _rup(e_h2h, NW * 128)
    h2e_pad = _rup(e_h2e, NW * 128)

    x_flat = jnp.transpose(x, (0, 2, 1, 3)).reshape(bs * n_era, -1)

    src = _mlp(p["fm_src"], [x_flat, era_latlons, p["era_trainable"]])
    dst = _mlp(p["fm_dst"], [h_latlons, p["h_trainable"]])
    e_fm = _mlp(p["fm_edge"], [_pad_rows(e2h_edge_attr, e2h_pad),
                               _pad_rows(p["e2h_trainable"], e2h_pad)])

    e2h_s = _pad_idx(e2h_edge_index[0], e2h_pad, 0)
    e2h_d = _pad_idx(e2h_edge_index[1], e2h_pad, 0)
    e2h_dscat = _pad_idx(e2h_edge_index[1], e2h_pad, 1 << 30)
    gs, gd = _sc_gather2(src, dst, e2h_s, e2h_d)
    m = _mlp(p["fm_msg"], [gs, gd, e_fm])
    agg = _sc_scatter(m, e2h_dscat, n_h, 1)
    x_latent = _mlp(p["fm_node"], [dst, agg], residual=True)

    e_pr = _mlp(p["proc_edge"], [_pad_rows(h2h_edge_attr, h2h_pad),
                                 _pad_rows(p["h2h_trainable"], h2h_pad)])
    h2h_s = _pad_idx(h2h_edge_index[0], h2h_pad, 0)
    h2h_d = _pad_idx(h2h_edge_index[1], h2h_pad, 0)
    h2h_dscat = _pad_idx(h2h_edge_index[1], h2h_pad, 1 << 30)

    gs, gd = _sc_gather2(x_latent, x_latent, h2h_s, h2h_d)
    m0, e_pr = _mlp(p["proc_msg_0"], [gs, gd, e_pr], e_new=True)
    agg = _sc_scatter(m0, h2h_dscat, n_h, 1)
    x_latent = _mlp(p["proc_node_0"], [x_latent, agg], residual=True)

    gs, gd = _sc_gather2(x_latent, x_latent, h2h_s, h2h_d)
    m1 = _mlp(p["proc_msg_1"], [gs, gd, e_pr])
    agg = _sc_scatter(m1, h2h_dscat, n_h, 1)
    x_latent = _mlp(p["proc_node_1"], [x_latent, agg], residual=True)

    e_bm = _mlp(p["bm_edge"], [_pad_rows(h2e_edge_attr, h2e_pad),
                               _pad_rows(p["h2e_trainable"], h2e_pad)])
    h2e_s = _pad_idx(h2e_edge_index[0], h2e_pad, 0)
    h2e_d = _pad_idx(h2e_edge_index[1], h2e_pad, 0)
    h2e_dscat = _pad_idx(h2e_edge_index[1], h2e_pad, 1 << 30)
    gs, gd = _sc_gather2(x_latent, src, h2e_s, h2e_d)
    m2 = _mlp(p["bm_msg"], [gs, gd, e_bm])
    agg = _sc_scatter(m2, h2e_dscat, n_era, 2)
    out = _mlp(p["bm_node"], [src, agg], residual=True,
               proj=(p["bm_out_W"], p["bm_out_b"]))
    return out.reshape(bs, n_era, -1)
